# initial kernel scaffold (unmeasured)
import jax
import jax.numpy as jnp
from jax import lax
from jax.experimental import pallas as pl
from jax.experimental.pallas import tpu as pltpu

BLK = 256
SEND_SLOTS = 8

_ANY = getattr(pltpu, "ANY", None)
if _ANY is None:
    _ANY = getattr(pltpu, "TPUMemorySpace", getattr(pltpu, "MemorySpace")).ANY
_CompilerParams = getattr(pltpu, "CompilerParams", None) or getattr(
    pltpu, "TPUCompilerParams"
)
_DeviceIdType = getattr(pl, "DeviceIdType", None) or getattr(pltpu, "DeviceIdType")
_sem_signal = getattr(pl, "semaphore_signal", None) or getattr(
    pltpu, "semaphore_signal"
)
_sem_wait = getattr(pl, "semaphore_wait", None) or getattr(pltpu, "semaphore_wait")


def kernel(O, Wo):
    B, S, H, D = O.shape
    K = H * D
    N = Wo.shape[1]
    R = S // 2
    NB = N // BLK

    O2 = O.reshape(S, K).astype(jnp.bfloat16)

    def body(o_ref, wo_ref, out_ref, wo_tiles, send_buf, recv_buf, po_stage,
             rb_stage, wb_stage, wo_sems, send_sems, recv_sems, po_sems,
             rb_sems, wb_sems):
        x = lax.axis_index("x")
        y = lax.axis_index("y")
        z = lax.axis_index("z")
        peer = (1 - x, y, z)
        is0 = x == 0

        barrier = pltpu.get_barrier_semaphore()
        _sem_signal(barrier, inc=1, device_id=peer,
                    device_id_type=_DeviceIdType.MESH)
        _sem_wait(barrier, 1)

        def wo_copy(t):
            return pltpu.make_async_copy(
                wo_ref.at[:, pl.ds(t * BLK, BLK)],
                wo_tiles.at[t % 2],
                wo_sems.at[t % 2],
            )

        def send_rdma(slot, t):
            return pltpu.make_async_remote_copy(
                src_ref=send_buf.at[slot],
                dst_ref=recv_buf.at[t],
                send_sem=send_sems.at[slot],
                recv_sem=recv_sems.at[t],
                device_id=peer,
                device_id_type=_DeviceIdType.MESH,
            )

        def po_copy(t):
            return pltpu.make_async_copy(
                po_stage.at[t % 2],
                out_ref.at[:, pl.ds(t * BLK, BLK)],
                po_sems.at[t % 2],
            )

        def rb_copy(c):
            return pltpu.make_async_copy(
                out_ref.at[:, pl.ds(c * BLK, BLK)],
                rb_stage.at[c % 2],
                rb_sems.at[c % 2],
            )

        def wb_copy(c):
            return pltpu.make_async_copy(
                wb_stage.at[c % 2],
                out_ref.at[:, pl.ds(c * BLK, BLK)],
                wb_sems.at[c % 2],
            )

        wo_copy(0).start()
        for t in range(NB):
            if t + 1 < NB:
                wo_copy(t + 1).start()
            wo_copy(t).wait()
            wt = wo_tiles[t % 2].astype(jnp.bfloat16)
            p = jnp.dot(o_ref[...], wt, preferred_element_type=jnp.float32)
            mine = jnp.where(is0, p[:R, :], p[R:, :])
            their = jnp.where(is0, p[R:, :], p[:R, :])

            s = t % SEND_SLOTS
            if t >= SEND_SLOTS:
                send_rdma(s, t).wait_send()
            send_buf[s] = their.astype(jnp.bfloat16)
            send_rdma(s, t).start()

            if t >= 2:
                po_copy(t - 2).wait()
            po_stage[t % 2] = mine.astype(jnp.bfloat16)
            po_copy(t).start()

        for s in range(SEND_SLOTS):
            send_rdma(s, NB - SEND_SLOTS + s).wait_send()
        po_copy(NB - 2).wait()
        po_copy(NB - 1).wait()

        rb_copy(0).start()
        for c in range(NB):
            if c + 1 < NB:
                rb_copy(c + 1).start()
            rb_copy(c).wait()
            send_rdma(0, c).wait_recv()
            if c >= 2:
                wb_copy(c - 2).wait()
            wb_stage[c % 2] = (
                rb_stage[c % 2].astype(jnp.float32)
                + recv_buf[c].astype(jnp.float32)
            ).astype(jnp.bfloat16)
            wb_copy(c).start()
        wb_copy(NB - 2).wait()
        wb_copy(NB - 1).wait()

    out = pl.pallas_call(
        body,
        out_shape=jax.ShapeDtypeStruct((R, N), jnp.bfloat16),
        in_specs=[
            pl.BlockSpec(memory_space=pltpu.VMEM),
            pl.BlockSpec(memory_space=_ANY),
        ],
        out_specs=pl.BlockSpec(memory_space=_ANY),
        scratch_shapes=[
            pltpu.VMEM((2, K, BLK), jnp.float32),
            pltpu.VMEM((SEND_SLOTS, R, BLK), jnp.bfloat16),
            pltpu.VMEM((NB, R, BLK), jnp.bfloat16),
            pltpu.VMEM((2, R, BLK), jnp.bfloat16),
            pltpu.VMEM((2, R, BLK), jnp.bfloat16),
            pltpu.VMEM((2, R, BLK), jnp.bfloat16),
            pltpu.SemaphoreType.DMA((2,)),
            pltpu.SemaphoreType.DMA((SEND_SLOTS,)),
            pltpu.SemaphoreType.DMA((NB,)),
            pltpu.SemaphoreType.DMA((2,)),
            pltpu.SemaphoreType.DMA((2,)),
            pltpu.SemaphoreType.DMA((2,)),
        ],
        compiler_params=_CompilerParams(collective_id=0),
    )(O2, Wo)

    return out.reshape(B, R, N)


# baseline (device time: 244557 ns/iter reference)
import jax
import jax.numpy as jnp
from jax import lax
from jax.experimental import pallas as pl
from jax.experimental.pallas import tpu as pltpu

BLK = 256

_CompilerParams = getattr(pltpu, "CompilerParams", None) or getattr(
    pltpu, "TPUCompilerParams"
)
_DeviceIdType = getattr(pl, "DeviceIdType", None) or getattr(pltpu, "DeviceIdType")
_sem_signal = getattr(pl, "semaphore_signal", None) or getattr(
    pltpu, "semaphore_signal"
)
_sem_wait = getattr(pl, "semaphore_wait", None) or getattr(pltpu, "semaphore_wait")
_HBM = pltpu.MemorySpace.HBM


def kernel(O, Wo):
    B, S, H, D = O.shape
    K = H * D
    N = Wo.shape[1]
    R = S // 2
    NB = N // BLK

    O2 = O.reshape(S, K).astype(jnp.bfloat16)

    def body(o_ref, wo_ref, out_mine_ref, out_recv_ref, send_buf,
             send_sems, recv_sems):
        t = pl.program_id(0)
        x = lax.axis_index("x")
        y = lax.axis_index("y")
        z = lax.axis_index("z")
        peer = (1 - x, y, z)
        is0 = x == 0

        @pl.when(t == 0)
        def _():
            barrier = pltpu.get_barrier_semaphore()
            _sem_signal(barrier, inc=1, device_id=peer,
                        device_id_type=_DeviceIdType.MESH)
            _sem_wait(barrier, 1)

        wt = wo_ref[...].astype(jnp.bfloat16)
        p = jnp.dot(o_ref[...], wt, preferred_element_type=jnp.float32)
        mine = jnp.where(is0, p[:R, :], p[R:, :])
        their = jnp.where(is0, p[R:, :], p[:R, :])

        out_mine_ref[...] = mine.astype(jnp.bfloat16)
        send_buf[t] = their.astype(jnp.bfloat16)

        def rdma(i):
            return pltpu.make_async_remote_copy(
                src_ref=send_buf.at[i],
                dst_ref=out_recv_ref.at[:, pl.ds(i * BLK, BLK)],
                send_sem=send_sems.at[i],
                recv_sem=recv_sems.at[i],
                device_id=peer,
                device_id_type=_DeviceIdType.MESH,
            )

        rdma(t).start()

        @pl.when(t == NB - 1)
        def _():
            for i in range(NB):
                rdma(i).wait_send()
            for i in range(NB):
                rdma(i).wait_recv()

    out_mine, out_recv = pl.pallas_call(
        body,
        grid=(NB,),
        out_shape=[
            jax.ShapeDtypeStruct((R, N), jnp.bfloat16),
            jax.ShapeDtypeStruct((R, N), jnp.bfloat16),
        ],
        in_specs=[
            pl.BlockSpec((S, K), lambda t: (0, 0)),
            pl.BlockSpec((K, BLK), lambda t: (0, t)),
        ],
        out_specs=[
            pl.BlockSpec((R, BLK), lambda t: (0, t)),
            pl.BlockSpec(memory_space=_HBM),
        ],
        scratch_shapes=[
            pltpu.VMEM((NB, R, BLK), jnp.bfloat16),
            pltpu.SemaphoreType.DMA((NB,)),
            pltpu.SemaphoreType.DMA((NB,)),
        ],
        compiler_params=_CompilerParams(
            dimension_semantics=("arbitrary",),
            collective_id=0,
            vmem_limit_bytes=60 * 1024 * 1024,
        ),
    )(O2, Wo)

    out = out_mine.astype(jnp.float32) + out_recv.astype(jnp.float32)
    return out.reshape(B, R, N)
